# baseline (speedup 1.0000x reference)
import functools

import jax
import jax.numpy as jnp
from jax import lax
from jax.experimental import pallas as pl
from jax.experimental.pallas import tpu as pltpu
from jax.experimental.pallas import tpu_sc as plsc

NUM_EMBEDDINGS = 1000000
EMBEDDING_DIM = 32

NC = 2
NS = 16
NW = NC * NS

TOTAL = 4096 * 26
RPW = TOTAL // NW
CHUNK = 128
NCHUNK = RPW // CHUNK

CPT = 62720
CPAD = CPT * NS
ZBUF = 6272


def _body(idx_hbm, table_hbm, out_emb, out_counts,
          idx_v, rows_v, ones_v, zbuf_v, counts_sh, gsem):
    c = lax.axis_index("c")
    s = lax.axis_index("s")
    wid = s * NC + c

    pltpu.sync_copy(idx_hbm.at[wid], idx_v.at[0])

    @pl.when(c == 0)
    def _zero():
        def fill(i, _):
            zbuf_v[pl.ds(i * 16, 16)] = jnp.zeros((16,), jnp.int32)
            return _
        lax.fori_loop(0, ZBUF // 16, fill, 0)

        def zcopy(k, _):
            pltpu.sync_copy(zbuf_v, counts_sh.at[pl.ds(s * CPT + k * ZBUF, ZBUF)])
            return _
        lax.fori_loop(0, CPT // ZBUF, zcopy, 0)

        def ofill(i, _):
            ones_v[pl.ds(i * 16, 16)] = jnp.ones((16,), jnp.int32)
            return _
        lax.fori_loop(0, CHUNK // 16, ofill, 0)

        pltpu.sync_copy(idx_hbm.at[wid + 1], idx_v.at[1])

        plsc.subcore_barrier()

        def scat(t, _):
            p = t // NCHUNK
            j = t - p * NCHUNK
            pltpu.sync_copy(ones_v, counts_sh.at[idx_v.at[p, j]], add=True)
            return _
        lax.fori_loop(0, 2 * NCHUNK, scat, 0)

        plsc.subcore_barrier()

        pltpu.sync_copy(counts_sh.at[pl.ds(s * CPT, CPT)],
                        out_counts.at[pl.ds(s * CPT, CPT)])

    base = wid * RPW

    def gat(j, _):
        pltpu.async_copy(table_hbm.at[idx_v.at[0, j]], rows_v, gsem).wait()
        pltpu.sync_copy(rows_v, out_emb.at[pl.ds(base + j * CHUNK, CHUNK)])
        return _
    lax.fori_loop(0, NCHUNK, gat, 0)


@jax.jit
def _run(idx, weight):
    mesh = plsc.VectorSubcoreMesh(core_axis_name="c", subcore_axis_name="s")
    fn = pl.kernel(
        _body,
        out_type=(
            jax.ShapeDtypeStruct((TOTAL, EMBEDDING_DIM), jnp.float32),
            jax.ShapeDtypeStruct((CPAD,), jnp.int32),
        ),
        mesh=mesh,
        compiler_params=pltpu.CompilerParams(use_tc_tiling_on_sc=False),
        scratch_types=(
            pltpu.VMEM((2, NCHUNK, CHUNK), jnp.int32),
            pltpu.VMEM((CHUNK, EMBEDDING_DIM), jnp.float32),
            pltpu.VMEM((CHUNK,), jnp.int32),
            pltpu.VMEM((ZBUF,), jnp.int32),
            pltpu.VMEM_SHARED((CPAD,), jnp.int32),
            pltpu.SemaphoreType.DMA,
        ),
    )
    return fn(idx, weight)


def kernel(input, weight):
    idx = input.reshape(NW, NCHUNK, CHUNK)
    emb_flat, counts_pad = _run(idx, weight)
    emb = emb_flat.reshape(input.shape + (EMBEDDING_DIM,))
    return emb, counts_pad[:NUM_EMBEDDINGS]


# speedup vs baseline: 1.0323x; 1.0323x over previous
import functools

import jax
import jax.numpy as jnp
from jax import lax
from jax.experimental import pallas as pl
from jax.experimental.pallas import tpu as pltpu
from jax.experimental.pallas import tpu_sc as plsc

NUM_EMBEDDINGS = 1000000
EMBEDDING_DIM = 32

NC = 2
NS = 16
NW = NC * NS

TOTAL = 4096 * 26
RPW = TOTAL // NW
CHUNK = 128
NCHUNK = RPW // CHUNK
CPB = 13
BROWS = CPB * CHUNK
NROUND = NCHUNK // CPB

CPT = 62720
CPAD = CPT * NS
ZBUF = 3136


def _body(idx_hbm, table_hbm, out_emb, out_counts,
          idx_v, rows_v, ones_v, zbuf_v, counts_sh,
          gsem, osem, csem, zsem):
    c = lax.axis_index("c")
    s = lax.axis_index("s")
    wid = s * NC + c
    cbase = wid * NCHUNK

    pltpu.sync_copy(idx_hbm.at[wid], idx_v.at[0])

    def gfire(r, b):
        def one(jj, _):
            j = r * CPB + jj
            pltpu.async_copy(table_hbm.at[idx_v.at[0, j]],
                             rows_v.at[b, jj], gsem)
            return _
        lax.fori_loop(0, CPB, one, 0)

    gfire(0, 0)

    @pl.when(c == 0)
    def _counts():
        def fill(i, _):
            zbuf_v[pl.ds(i * 16, 16)] = jnp.zeros((16,), jnp.int32)
            return _
        lax.fori_loop(0, ZBUF // 16, fill, 0)

        def zfire(k, _):
            pltpu.sync_copy(zbuf_v,
                            counts_sh.at[pl.ds(s * CPT + k * ZBUF, ZBUF)])
            return _
        lax.fori_loop(0, CPT // ZBUF, zfire, 0)

        def ofill(i, _):
            ones_v[pl.ds(i * 16, 16)] = jnp.ones((16,), jnp.int32)
            return _
        lax.fori_loop(0, CHUNK // 16, ofill, 0)

        pltpu.sync_copy(idx_hbm.at[wid + 1], idx_v.at[1])

        plsc.subcore_barrier()

        def scat(t, _):
            p = t // NCHUNK
            j = t - p * NCHUNK
            pltpu.sync_copy(ones_v.at[pl.ds(0, CHUNK)],
                            counts_sh.at[idx_v.at[p, j]], add=True)
            return _
        lax.fori_loop(0, 2 * NCHUNK, scat, 0)

        plsc.subcore_barrier()

        pltpu.sync_copy(counts_sh.at[pl.ds(s * CPT, CPT)],
                        out_counts.at[pl.ds(s * CPT, CPT)])

    for r in range(NROUND):
        b = 0
        dst = out_emb.at[pl.ds(cbase + r * CPB, CPB)]

        def gdrain(jj, _, r=r, b=b):
            j = r * CPB + jj
            pltpu.make_async_copy(table_hbm.at[idx_v.at[0, j]],
                                  rows_v.at[b, jj], gsem).wait()
            return _
        lax.fori_loop(0, CPB, gdrain, 0)
        pltpu.sync_copy(rows_v.at[b], dst)
        if r + 1 < NROUND:
            gfire(r + 1, b)


@jax.jit
def _run(idx, weight):
    mesh = plsc.VectorSubcoreMesh(core_axis_name="c", subcore_axis_name="s")
    fn = pl.kernel(
        _body,
        out_type=(
            jax.ShapeDtypeStruct((TOTAL // CHUNK, CHUNK, EMBEDDING_DIM),
                                 jnp.float32),
            jax.ShapeDtypeStruct((CPAD,), jnp.int32),
        ),
        mesh=mesh,
        compiler_params=pltpu.CompilerParams(use_tc_tiling_on_sc=False),
        scratch_types=(
            pltpu.VMEM((2, NCHUNK, CHUNK), jnp.int32),
            pltpu.VMEM((1, CPB, CHUNK, EMBEDDING_DIM), jnp.float32),
            pltpu.VMEM((128,), jnp.int32),
            pltpu.VMEM((ZBUF,), jnp.int32),
            pltpu.VMEM_SHARED((CPAD,), jnp.int32),
            pltpu.SemaphoreType.DMA,
            pltpu.SemaphoreType.DMA,
            pltpu.SemaphoreType.DMA,
            pltpu.SemaphoreType.DMA,
        ),
    )
    return fn(idx, weight)


def kernel(input, weight):
    idx = input.reshape(NW, NCHUNK, CHUNK)
    emb_flat, counts_pad = _run(idx, weight)
    emb = emb_flat.reshape(input.shape + (EMBEDDING_DIM,))
    return emb, counts_pad[:NUM_EMBEDDINGS]


# speedup vs baseline: 1.0373x; 1.0049x over previous
import functools

import jax
import jax.numpy as jnp
from jax import lax
from jax.experimental import pallas as pl
from jax.experimental.pallas import tpu as pltpu
from jax.experimental.pallas import tpu_sc as plsc

NUM_EMBEDDINGS = 1000000
EMBEDDING_DIM = 32

NC = 2
NS = 16
NW = NC * NS

TOTAL = 4096 * 26
RPW = TOTAL // NW
CHUNK = 128
NCHUNK = RPW // CHUNK
CPB = 13
BROWS = CPB * CHUNK
NROUND = NCHUNK // CPB

CPT = 62720
CPAD = CPT * NS
CPT_LAST = NUM_EMBEDDINGS - (NS - 1) * CPT
ZBUF = 3136


def _body(idx_hbm, table_hbm, out_emb, out_counts,
          idx_v, rows_v, ones_v, zbuf_v, counts_sh,
          gsem, osem, csem, zsem):
    c = lax.axis_index("c")
    s = lax.axis_index("s")
    wid = s * NC + c
    cbase = wid * NCHUNK

    pltpu.sync_copy(idx_hbm.at[wid], idx_v.at[0])

    def gfire(r, b):
        def one(jj, _):
            j = r * CPB + jj
            pltpu.async_copy(table_hbm.at[idx_v.at[0, j]],
                             rows_v.at[b, jj], gsem)
            return _
        lax.fori_loop(0, CPB, one, 0)

    gfire(0, 0)

    @pl.when(c == 0)
    def _counts():
        def fill(i, _):
            zbuf_v[pl.ds(i * 16, 16)] = jnp.zeros((16,), jnp.int32)
            return _
        lax.fori_loop(0, ZBUF // 16, fill, 0)

        def zfire(k, _):
            pltpu.sync_copy(zbuf_v,
                            counts_sh.at[pl.ds(s * CPT + k * ZBUF, ZBUF)])
            return _
        lax.fori_loop(0, CPT // ZBUF, zfire, 0)

        def ofill(i, _):
            ones_v[pl.ds(i * 16, 16)] = jnp.ones((16,), jnp.int32)
            return _
        lax.fori_loop(0, CHUNK // 16, ofill, 0)

        pltpu.sync_copy(idx_hbm.at[wid + 1], idx_v.at[1])

        plsc.subcore_barrier()

        def scat(t, _):
            p = t // NCHUNK
            j = t - p * NCHUNK
            pltpu.sync_copy(ones_v.at[pl.ds(0, CHUNK)],
                            counts_sh.at[idx_v.at[p, j]], add=True)
            return _
        lax.fori_loop(0, 2 * NCHUNK, scat, 0)

        plsc.subcore_barrier()

        @pl.when(s < NS - 1)
        def _full():
            pltpu.sync_copy(counts_sh.at[pl.ds(s * CPT, CPT)],
                            out_counts.at[pl.ds(s * CPT, CPT)])

        @pl.when(s == NS - 1)
        def _last():
            pltpu.sync_copy(counts_sh.at[pl.ds(s * CPT, CPT_LAST)],
                            out_counts.at[pl.ds(s * CPT, CPT_LAST)])

    for r in range(NROUND):
        b = 0
        dst = out_emb.at[pl.ds(cbase + r * CPB, CPB)]

        def gdrain(jj, _, r=r, b=b):
            j = r * CPB + jj
            pltpu.make_async_copy(table_hbm.at[idx_v.at[0, j]],
                                  rows_v.at[b, jj], gsem).wait()
            return _
        lax.fori_loop(0, CPB, gdrain, 0)
        pltpu.sync_copy(rows_v.at[b], dst)
        if r + 1 < NROUND:
            gfire(r + 1, b)


@jax.jit
def _run(idx, weight):
    mesh = plsc.VectorSubcoreMesh(core_axis_name="c", subcore_axis_name="s")
    fn = pl.kernel(
        _body,
        out_type=(
            jax.ShapeDtypeStruct((TOTAL // CHUNK, CHUNK, EMBEDDING_DIM),
                                 jnp.float32),
            jax.ShapeDtypeStruct((NUM_EMBEDDINGS,), jnp.int32),
        ),
        mesh=mesh,
        compiler_params=pltpu.CompilerParams(use_tc_tiling_on_sc=False),
        scratch_types=(
            pltpu.VMEM((2, NCHUNK, CHUNK), jnp.int32),
            pltpu.VMEM((1, CPB, CHUNK, EMBEDDING_DIM), jnp.float32),
            pltpu.VMEM((128,), jnp.int32),
            pltpu.VMEM((ZBUF,), jnp.int32),
            pltpu.VMEM_SHARED((CPAD,), jnp.int32),
            pltpu.SemaphoreType.DMA,
            pltpu.SemaphoreType.DMA,
            pltpu.SemaphoreType.DMA,
            pltpu.SemaphoreType.DMA,
        ),
    )
    return fn(idx, weight)


def kernel(input, weight):
    idx = input.reshape(NW, NCHUNK, CHUNK)
    wlin = jax.lax.optimization_barrier(
        weight.reshape(NUM_EMBEDDINGS * EMBEDDING_DIM))
    w2 = wlin.reshape(NUM_EMBEDDINGS, EMBEDDING_DIM)
    emb_flat, counts = _run(idx, w2)
    emb = emb_flat.reshape(input.shape + (EMBEDDING_DIM,))
    return emb, counts


# speedup vs baseline: 1.1516x; 1.1102x over previous
import functools

import jax
import jax.numpy as jnp
from jax import lax
from jax.experimental import pallas as pl
from jax.experimental.pallas import tpu as pltpu
from jax.experimental.pallas import tpu_sc as plsc

NUM_EMBEDDINGS = 1000000
EMBEDDING_DIM = 32

NC = 2
NS = 16
NW = NC * NS

TOTAL = 4096 * 26
RPW = TOTAL // NW
CHUNK = 128
NCHUNK = RPW // CHUNK
NBLK = 250880

CPT = 62720
CPAD = CPT * NS
CPT_LAST = NUM_EMBEDDINGS - (NS - 1) * CPT
ZBUF = 3136


def _body(idx_hbm, table_hbm, out_emb, out_counts,
          idx_v, idxq_v, big_v, comp_v, ones_v, zbuf_v, counts_sh,
          gsem, osem, csem, zsem):
    c = lax.axis_index("c")
    s = lax.axis_index("s")
    wid = s * NC + c
    cbase = wid * NCHUNK

    pltpu.sync_copy(idx_hbm.at[wid], idx_v.at[0])

    def qfill(t, _):
        jj = t // 8
        g = t - jj * 8
        v = idx_v[0, jj, pl.ds(g * 16, 16)]
        idxq_v[jj, pl.ds(g * 16, 16)] = v - (v // NBLK) * NBLK
        return _
    lax.fori_loop(0, NCHUNK * 8, qfill, 0)

    def gfire(j, b):
        pltpu.async_copy(table_hbm.at[idxq_v.at[j]], big_v.at[b], gsem)

    gfire(0, 0)

    @pl.when(c == 0)
    def _counts():
        def fill(i, _):
            zbuf_v[pl.ds(i * 16, 16)] = jnp.zeros((16,), jnp.int32)
            return _
        lax.fori_loop(0, ZBUF // 16, fill, 0)

        def zfire(k, _):
            pltpu.sync_copy(zbuf_v,
                            counts_sh.at[pl.ds(s * CPT + k * ZBUF, ZBUF)])
            return _
        lax.fori_loop(0, CPT // ZBUF, zfire, 0)

        def ofill(i, _):
            ones_v[pl.ds(i * 16, 16)] = jnp.ones((16,), jnp.int32)
            return _
        lax.fori_loop(0, CHUNK // 16, ofill, 0)

        pltpu.sync_copy(idx_hbm.at[wid + 1], idx_v.at[1])

        plsc.subcore_barrier()

        def scat(t, _):
            p = t // NCHUNK
            j = t - p * NCHUNK
            pltpu.sync_copy(ones_v.at[pl.ds(0, CHUNK)],
                            counts_sh.at[idx_v.at[p, j]], add=True)
            return _
        lax.fori_loop(0, 2 * NCHUNK, scat, 0)

        plsc.subcore_barrier()

        @pl.when(s < NS - 1)
        def _full():
            pltpu.sync_copy(counts_sh.at[pl.ds(s * CPT, CPT)],
                            out_counts.at[pl.ds(s * CPT, CPT)])

        @pl.when(s == NS - 1)
        def _last():
            pltpu.sync_copy(counts_sh.at[pl.ds(s * CPT, CPT_LAST)],
                            out_counts.at[pl.ds(s * CPT, CPT_LAST)])

    lane = jax.lax.iota(jnp.int32, 16)
    for j in range(NCHUNK):
        b = j % 2
        if j + 1 < NCHUNK:
            gfire(j + 1, 1 - b)
        pltpu.make_async_copy(table_hbm.at[idxq_v.at[j]], big_v.at[b],
                              gsem).wait()
        if j >= 2:
            pltpu.make_async_copy(comp_v.at[b],
                                  out_emb.at[cbase + j - 2],
                                  osem if b == 0 else zsem).wait()

        bb = jnp.full((16,), b, jnp.int32)
        def extract(g, _, j=j, b=b, bb=bb):
            idxv = idx_v[0, j, pl.ds(g * 16, 16)]
            off = (idxv // NBLK) * 32
            rows = g * 16 + lane
            def inner(ci, _):
                v = plsc.load_gather(big_v, [bb, rows, off + ci])
                plsc.store_scatter(comp_v, [bb, rows,
                                            jnp.full((16,), 0, jnp.int32) + ci],
                                   v)
                return _
            lax.fori_loop(0, 32, inner, 0)
            return _
        lax.fori_loop(0, 8, extract, 0)
        pltpu.async_copy(comp_v.at[b], out_emb.at[cbase + j],
                         osem if b == 0 else zsem)
    for j in (NCHUNK - 2, NCHUNK - 1):
        b = j % 2
        pltpu.make_async_copy(comp_v.at[b], out_emb.at[cbase + j],
                              osem if b == 0 else zsem).wait()


@jax.jit
def _run(idx, weight):
    mesh = plsc.VectorSubcoreMesh(core_axis_name="c", subcore_axis_name="s")
    fn = pl.kernel(
        _body,
        out_type=(
            jax.ShapeDtypeStruct((TOTAL // CHUNK, CHUNK, EMBEDDING_DIM),
                                 jnp.float32),
            jax.ShapeDtypeStruct((NUM_EMBEDDINGS,), jnp.int32),
        ),
        mesh=mesh,
        compiler_params=pltpu.CompilerParams(use_tc_tiling_on_sc=False,
                                             needs_layout_passes=False),
        scratch_types=(
            pltpu.VMEM((2, NCHUNK, CHUNK), jnp.int32),
            pltpu.VMEM((NCHUNK, CHUNK), jnp.int32),
            pltpu.VMEM((2, CHUNK, 128), jnp.float32),
            pltpu.VMEM((2, CHUNK, EMBEDDING_DIM), jnp.float32),
            pltpu.VMEM((128,), jnp.int32),
            pltpu.VMEM((ZBUF,), jnp.int32),
            pltpu.VMEM_SHARED((CPAD,), jnp.int32),
            pltpu.SemaphoreType.DMA,
            pltpu.SemaphoreType.DMA,
            pltpu.SemaphoreType.DMA,
            pltpu.SemaphoreType.DMA,
        ),
    )
    return fn(idx, weight)


TW = 1024
TGRID = NBLK // TW


def _relayout_body(i0, i1, i2, i3, o_ref):
    o_ref[...] = jnp.concatenate(
        [i0[...].T, i1[...].T, i2[...].T, i3[...].T], axis=1)


def _relayout_tc(wt):
    last = NUM_EMBEDDINGS // TW
    specs = [
        pl.BlockSpec((EMBEDDING_DIM, TW),
                     lambda g, s=s: (0, jnp.minimum(g + s * TGRID, last)))
        for s in range(4)
    ]
    return pl.pallas_call(
        _relayout_body,
        grid=(TGRID,),
        in_specs=[specs[0], specs[1], specs[2], specs[3]],
        out_specs=pl.BlockSpec((TW, 4 * EMBEDDING_DIM), lambda g: (g, 0)),
        out_shape=jax.ShapeDtypeStruct(
            (NBLK, 4 * EMBEDDING_DIM), jnp.float32),
    )(wt, wt, wt, wt)


def kernel(input, weight):
    idx = input.reshape(NW, NCHUNK, CHUNK)
    w128 = _relayout_tc(weight.T)
    emb_flat, counts = _run(idx, w128)
    emb = emb_flat.reshape(input.shape + (EMBEDDING_DIM,))
    return emb, counts


# speedup vs baseline: 1.4167x; 1.2301x over previous
import functools

import jax
import jax.numpy as jnp
from jax import lax
from jax.experimental import pallas as pl
from jax.experimental.pallas import tpu as pltpu
from jax.experimental.pallas import tpu_sc as plsc

NUM_EMBEDDINGS = 1000000
EMBEDDING_DIM = 32

NC = 2
NS = 16
NW = NC * NS

TOTAL = 4096 * 26
RPW = TOTAL // NW
CHUNK = 128
NCHUNK = RPW // CHUNK
NBLK = 250880

CPT = 62720
CPAD = CPT * NS
CPT_LAST = NUM_EMBEDDINGS - (NS - 1) * CPT
ZBUF = 3136


def _body(idx_hbm, table_hbm, out_emb, out_counts,
          idx_v, idxq_v, big_v, comp_v, ones_v, zbuf_v, counts_sh,
          gsem, osem, csem, zsem):
    c = lax.axis_index("c")
    s = lax.axis_index("s")
    wid = s * NC + c
    cbase = wid * NCHUNK

    pltpu.sync_copy(idx_hbm.at[wid], idx_v.at[0])

    def qfill(t, _):
        jj = t // 8
        g = t - jj * 8
        v = idx_v[0, jj, pl.ds(g * 16, 16)]
        idxq_v[jj, pl.ds(g * 16, 16)] = v - (v // NBLK) * NBLK
        return _
    lax.fori_loop(0, NCHUNK * 8, qfill, 0)

    def gfire(j, b):
        pltpu.async_copy(table_hbm.at[idxq_v.at[j]], big_v.at[b], gsem)

    gfire(0, 0)

    @pl.when(c == 0)
    def _counts():
        def fill(i, _):
            zbuf_v[pl.ds(i * 16, 16)] = jnp.zeros((16,), jnp.int32)
            return _
        lax.fori_loop(0, ZBUF // 16, fill, 0)

        def zfire(k, _):
            pltpu.sync_copy(zbuf_v,
                            counts_sh.at[pl.ds(s * CPT + k * ZBUF, ZBUF)])
            return _
        lax.fori_loop(0, CPT // ZBUF, zfire, 0)

        def ofill(i, _):
            ones_v[pl.ds(i * 16, 16)] = jnp.ones((16,), jnp.int32)
            return _
        lax.fori_loop(0, CHUNK // 16, ofill, 0)

        pltpu.sync_copy(idx_hbm.at[wid + 1], idx_v.at[1])

        plsc.subcore_barrier()

        def scat(t, _):
            p = t // NCHUNK
            j = t - p * NCHUNK
            pltpu.sync_copy(ones_v.at[pl.ds(0, CHUNK)],
                            counts_sh.at[idx_v.at[p, j]], add=True)
            return _
        lax.fori_loop(0, 2 * NCHUNK, scat, 0)

        plsc.subcore_barrier()

        @pl.when(s < NS - 1)
        def _full():
            pltpu.sync_copy(counts_sh.at[pl.ds(s * CPT, CPT)],
                            out_counts.at[pl.ds(s * CPT, CPT)])

        @pl.when(s == NS - 1)
        def _last():
            pltpu.sync_copy(counts_sh.at[pl.ds(s * CPT, CPT_LAST)],
                            out_counts.at[pl.ds(s * CPT, CPT_LAST)])

    lane = jax.lax.iota(jnp.int32, 16)
    for j in range(NCHUNK):
        b = j % 2
        if j + 1 < NCHUNK:
            gfire(j + 1, 1 - b)
        pltpu.make_async_copy(table_hbm.at[idxq_v.at[j]], big_v.at[b],
                              gsem).wait()
        if j >= 2:
            pltpu.make_async_copy(comp_v.at[b],
                                  out_emb.at[cbase + j - 2],
                                  osem if b == 0 else zsem).wait()

        bb = jnp.full((16,), b, jnp.int32)
        def extract(g, _, j=j, b=b, bb=bb):
            idxv = idx_v[0, j, pl.ds(g * 16, 16)]
            off = (idxv // NBLK) * 32
            rows = g * 16 + lane
            def inner(ci, _):
                v = plsc.load_gather(big_v, [bb, rows, off + ci])
                plsc.store_scatter(comp_v, [bb, rows,
                                            jnp.full((16,), 0, jnp.int32) + ci],
                                   v)
                return _
            lax.fori_loop(0, 32, inner, 0)
            return _
        lax.fori_loop(0, 8, extract, 0)
        pltpu.async_copy(comp_v.at[b], out_emb.at[cbase + j],
                         osem if b == 0 else zsem)
    for j in (NCHUNK - 2, NCHUNK - 1):
        b = j % 2
        pltpu.make_async_copy(comp_v.at[b], out_emb.at[cbase + j],
                              osem if b == 0 else zsem).wait()


@jax.jit
def _run(idx, weight):
    mesh = plsc.VectorSubcoreMesh(core_axis_name="c", subcore_axis_name="s")
    fn = pl.kernel(
        _body,
        out_type=(
            jax.ShapeDtypeStruct((TOTAL // CHUNK, CHUNK, EMBEDDING_DIM),
                                 jnp.float32),
            jax.ShapeDtypeStruct((NUM_EMBEDDINGS,), jnp.int32),
        ),
        mesh=mesh,
        compiler_params=pltpu.CompilerParams(use_tc_tiling_on_sc=False,
                                             needs_layout_passes=False),
        scratch_types=(
            pltpu.VMEM((2, NCHUNK, CHUNK), jnp.int32),
            pltpu.VMEM((NCHUNK, CHUNK), jnp.int32),
            pltpu.VMEM((2, CHUNK, 128), jnp.float32),
            pltpu.VMEM((2, CHUNK, EMBEDDING_DIM), jnp.float32),
            pltpu.VMEM((128,), jnp.int32),
            pltpu.VMEM((ZBUF,), jnp.int32),
            pltpu.VMEM_SHARED((CPAD,), jnp.int32),
            pltpu.SemaphoreType.DMA,
            pltpu.SemaphoreType.DMA,
            pltpu.SemaphoreType.DMA,
            pltpu.SemaphoreType.DMA,
        ),
    )
    return fn(idx, weight)


TW = 1024
TGRID = NBLK // TW


def _relayout_body(i0, i1, i2, i3, o_ref):
    o_ref[...] = jnp.concatenate(
        [i0[...], i1[...], i2[...], i3[...]], axis=0).T


def _relayout_tc(wt):
    last = NUM_EMBEDDINGS // TW
    specs = [
        pl.BlockSpec((EMBEDDING_DIM, TW),
                     lambda g, s=s: (0, jnp.minimum(g + s * TGRID, last)))
        for s in range(4)
    ]
    return pl.pallas_call(
        _relayout_body,
        grid=(TGRID,),
        in_specs=[specs[0], specs[1], specs[2], specs[3]],
        out_specs=pl.BlockSpec((TW, 4 * EMBEDDING_DIM), lambda g: (g, 0)),
        out_shape=jax.ShapeDtypeStruct(
            (NBLK, 4 * EMBEDDING_DIM), jnp.float32),
    )(wt, wt, wt, wt)


def kernel(input, weight):
    idx = input.reshape(NW, NCHUNK, CHUNK)
    w128 = _relayout_tc(weight.T)
    emb_flat, counts = _run(idx, w128)
    emb = emb_flat.reshape(input.shape + (EMBEDDING_DIM,))
    return emb, counts


# speedup vs baseline: 1.8166x; 1.2823x over previous
import functools

import jax
import jax.numpy as jnp
from jax import lax
from jax.experimental import pallas as pl
from jax.experimental.pallas import tpu as pltpu
from jax.experimental.pallas import tpu_sc as plsc

NUM_EMBEDDINGS = 1000000
EMBEDDING_DIM = 32

NC = 2
NS = 16
NW = NC * NS

TOTAL = 4096 * 26
RPW = TOTAL // NW
CHUNK = 128
NCHUNK = RPW // CHUNK
NBLK = 253952

CPT = 62720
CPAD = CPT * NS
CPT_LAST = NUM_EMBEDDINGS - (NS - 1) * CPT
ZBUF = 3136


def _body(idx_hbm, table_hbm, out_emb, out_counts,
          idx_v, idxq_v, big_v, comp_v, ones_v, zbuf_v, counts_sh,
          gsem, osem, csem, zsem):
    c = lax.axis_index("c")
    s = lax.axis_index("s")
    wid = s * NC + c
    cbase = wid * NCHUNK

    pltpu.sync_copy(idx_hbm.at[wid], idx_v.at[0])

    def qfill(t, _):
        jj = t // 8
        g = t - jj * 8
        v = idx_v[0, jj, pl.ds(g * 16, 16)]
        idxq_v[jj, pl.ds(g * 16, 16)] = v - (v // NBLK) * NBLK
        return _
    lax.fori_loop(0, NCHUNK * 8, qfill, 0)

    def gfire(j, b):
        pltpu.async_copy(table_hbm.at[idxq_v.at[j]], big_v.at[b], gsem)

    gfire(0, 0)

    @pl.when(c == 0)
    def _counts():
        def fill(i, _):
            zbuf_v[pl.ds(i * 16, 16)] = jnp.zeros((16,), jnp.int32)
            return _
        lax.fori_loop(0, ZBUF // 16, fill, 0)

        def zfire(k, _):
            pltpu.sync_copy(zbuf_v,
                            counts_sh.at[pl.ds(s * CPT + k * ZBUF, ZBUF)])
            return _
        lax.fori_loop(0, CPT // ZBUF, zfire, 0)

        def ofill(i, _):
            ones_v[pl.ds(i * 16, 16)] = jnp.ones((16,), jnp.int32)
            return _
        lax.fori_loop(0, CHUNK // 16, ofill, 0)

        pltpu.sync_copy(idx_hbm.at[wid + 1], idx_v.at[1])

        plsc.subcore_barrier()

        def scat(t, _):
            p = t // NCHUNK
            j = t - p * NCHUNK
            pltpu.sync_copy(ones_v.at[pl.ds(0, CHUNK)],
                            counts_sh.at[idx_v.at[p, j]], add=True)
            return _
        lax.fori_loop(0, 2 * NCHUNK, scat, 0)

        plsc.subcore_barrier()

        @pl.when(s < NS - 1)
        def _full():
            pltpu.sync_copy(counts_sh.at[pl.ds(s * CPT, CPT)],
                            out_counts.at[pl.ds(s * CPT, CPT)])

        @pl.when(s == NS - 1)
        def _last():
            pltpu.sync_copy(counts_sh.at[pl.ds(s * CPT, CPT_LAST)],
                            out_counts.at[pl.ds(s * CPT, CPT_LAST)])

    lane = jax.lax.iota(jnp.int32, 16)
    for j in range(NCHUNK):
        b = j % 2
        if j + 1 < NCHUNK:
            gfire(j + 1, 1 - b)
        pltpu.make_async_copy(table_hbm.at[idxq_v.at[j]], big_v.at[b],
                              gsem).wait()
        if j >= 2:
            pltpu.make_async_copy(comp_v.at[b],
                                  out_emb.at[cbase + j - 2],
                                  osem if b == 0 else zsem).wait()

        bb = jnp.full((16,), b, jnp.int32)
        def extract(g, _, j=j, b=b, bb=bb):
            idxv = idx_v[0, j, pl.ds(g * 16, 16)]
            off = (idxv // NBLK) * 32
            rows = g * 16 + lane
            def inner(ci, _):
                v = plsc.load_gather(big_v, [bb, rows, off + ci])
                plsc.store_scatter(comp_v, [bb, rows,
                                            jnp.full((16,), 0, jnp.int32) + ci],
                                   v)
                return _
            lax.fori_loop(0, 32, inner, 0)
            return _
        lax.fori_loop(0, 8, extract, 0)
        pltpu.async_copy(comp_v.at[b], out_emb.at[cbase + j],
                         osem if b == 0 else zsem)
    for j in (NCHUNK - 2, NCHUNK - 1):
        b = j % 2
        pltpu.make_async_copy(comp_v.at[b], out_emb.at[cbase + j],
                              osem if b == 0 else zsem).wait()


@jax.jit
def _run(idx, weight):
    mesh = plsc.VectorSubcoreMesh(core_axis_name="c", subcore_axis_name="s")
    fn = pl.kernel(
        _body,
        out_type=(
            jax.ShapeDtypeStruct((TOTAL // CHUNK, CHUNK, EMBEDDING_DIM),
                                 jnp.float32),
            jax.ShapeDtypeStruct((NUM_EMBEDDINGS,), jnp.int32),
        ),
        mesh=mesh,
        compiler_params=pltpu.CompilerParams(use_tc_tiling_on_sc=False,
                                             needs_layout_passes=False),
        scratch_types=(
            pltpu.VMEM((2, NCHUNK, CHUNK), jnp.int32),
            pltpu.VMEM((NCHUNK, CHUNK), jnp.int32),
            pltpu.VMEM((2, CHUNK, 128), jnp.float32),
            pltpu.VMEM((2, CHUNK, EMBEDDING_DIM), jnp.float32),
            pltpu.VMEM((128,), jnp.int32),
            pltpu.VMEM((ZBUF,), jnp.int32),
            pltpu.VMEM_SHARED((CPAD,), jnp.int32),
            pltpu.SemaphoreType.DMA,
            pltpu.SemaphoreType.DMA,
            pltpu.SemaphoreType.DMA,
            pltpu.SemaphoreType.DMA,
        ),
    )
    return fn(idx, weight)


TW = 4096
TGRID = NBLK // TW


def _relayout_body(i0, i1, i2, i3, o_ref):
    o_ref[...] = jnp.concatenate(
        [i0[...], i1[...], i2[...], i3[...]], axis=0).T


def _relayout_tc(wt):
    last = NUM_EMBEDDINGS // TW
    specs = [
        pl.BlockSpec((EMBEDDING_DIM, TW),
                     lambda g, s=s: (0, jnp.minimum(g + s * TGRID, last)))
        for s in range(4)
    ]
    return pl.pallas_call(
        _relayout_body,
        grid=(TGRID,),
        in_specs=[specs[0], specs[1], specs[2], specs[3]],
        out_specs=pl.BlockSpec((TW, 4 * EMBEDDING_DIM), lambda g: (g, 0)),
        out_shape=jax.ShapeDtypeStruct(
            (NBLK, 4 * EMBEDDING_DIM), jnp.float32),
    )(wt, wt, wt, wt)


def kernel(input, weight):
    idx = input.reshape(NW, NCHUNK, CHUNK)
    w128 = _relayout_tc(weight.T)
    emb_flat, counts = _run(idx, w128)
    emb = emb_flat.reshape(input.shape + (EMBEDDING_DIM,))
    return emb, counts


# speedup vs baseline: 1.8987x; 1.0452x over previous
import functools

import jax
import jax.numpy as jnp
from jax import lax
from jax.experimental import pallas as pl
from jax.experimental.pallas import tpu as pltpu
from jax.experimental.pallas import tpu_sc as plsc

NUM_EMBEDDINGS = 1000000
EMBEDDING_DIM = 32

NC = 2
NS = 16
NW = NC * NS

TOTAL = 4096 * 26
RPW = TOTAL // NW
CHUNK = 128
NCHUNK = RPW // CHUNK
NBLK = 253952

CPT = 62720
CPAD = CPT * NS
CPT_LAST = NUM_EMBEDDINGS - (NS - 1) * CPT
ZBUF = 3136


def _body(idx_hbm, table_hbm, out_emb, out_counts,
          idx_v, idxq_v, big_v, comp_v, ones_v, zbuf_v, counts_sh,
          gsem, osem, csem, zsem):
    c = lax.axis_index("c")
    s = lax.axis_index("s")
    wid = s * NC + c
    cbase = wid * NCHUNK

    pltpu.sync_copy(idx_hbm.at[wid], idx_v.at[0])

    def qfill(t, _):
        jj = t // 8
        g = t - jj * 8
        v = idx_v[0, jj, pl.ds(g * 16, 16)]
        idxq_v[jj, pl.ds(g * 16, 16)] = v - (v // NBLK) * NBLK
        return _
    lax.fori_loop(0, NCHUNK * 8, qfill, 0)

    def gfire(j, b):
        pltpu.async_copy(table_hbm.at[idxq_v.at[j]], big_v.at[b], gsem)

    gfire(0, 0)

    @pl.when(c == 0)
    def _counts():
        def fill(i, _):
            zbuf_v[pl.ds(i * 16, 16)] = jnp.zeros((16,), jnp.int32)
            return _
        lax.fori_loop(0, ZBUF // 16, fill, 0)

        def zfire(k, _):
            pltpu.sync_copy(zbuf_v,
                            counts_sh.at[pl.ds(s * CPT + k * ZBUF, ZBUF)])
            return _
        lax.fori_loop(0, CPT // ZBUF, zfire, 0)

        def ofill(i, _):
            ones_v[pl.ds(i * 16, 16)] = jnp.ones((16,), jnp.int32)
            return _
        lax.fori_loop(0, CHUNK // 16, ofill, 0)

        pltpu.sync_copy(idx_hbm.at[wid + 1], idx_v.at[1])

        plsc.subcore_barrier()

        def scat(t, _):
            p = t // NCHUNK
            j = t - p * NCHUNK
            pltpu.sync_copy(ones_v.at[pl.ds(0, CHUNK)],
                            counts_sh.at[idx_v.at[p, j]], add=True)
            return _
        lax.fori_loop(0, 2 * NCHUNK, scat, 0)

        plsc.subcore_barrier()

        @pl.when(s < NS - 1)
        def _full():
            pltpu.sync_copy(counts_sh.at[pl.ds(s * CPT, CPT)],
                            out_counts.at[pl.ds(s * CPT, CPT)])

        @pl.when(s == NS - 1)
        def _last():
            pltpu.sync_copy(counts_sh.at[pl.ds(s * CPT, CPT_LAST)],
                            out_counts.at[pl.ds(s * CPT, CPT_LAST)])

    lane = jax.lax.iota(jnp.int32, 16)
    for j in range(NCHUNK):
        b = j % 2
        if j + 1 < NCHUNK:
            gfire(j + 1, 1 - b)
        pltpu.make_async_copy(table_hbm.at[idxq_v.at[j]], big_v.at[b],
                              gsem).wait()
        if j >= 2:
            pltpu.make_async_copy(comp_v.at[b],
                                  out_emb.at[cbase + j - 2],
                                  osem if b == 0 else zsem).wait()

        bb = jnp.full((16,), b, jnp.int32)
        def extract(g, _, j=j, b=b, bb=bb):
            idxv = idx_v[0, j, pl.ds(g * 16, 16)]
            off = (idxv // NBLK) * 32
            rows = g * 16 + lane
            def inner(ci, _):
                v = plsc.load_gather(big_v, [bb, rows, off + ci])
                plsc.store_scatter(comp_v, [bb, rows,
                                            jnp.full((16,), 0, jnp.int32) + ci],
                                   v)
                return _
            lax.fori_loop(0, 32, inner, 0)
            return _
        lax.fori_loop(0, 8, extract, 0)
        pltpu.async_copy(comp_v.at[b], out_emb.at[cbase + j],
                         osem if b == 0 else zsem)
    for j in (NCHUNK - 2, NCHUNK - 1):
        b = j % 2
        pltpu.make_async_copy(comp_v.at[b], out_emb.at[cbase + j],
                              osem if b == 0 else zsem).wait()


@jax.jit
def _run(idx, weight):
    mesh = plsc.VectorSubcoreMesh(core_axis_name="c", subcore_axis_name="s")
    fn = pl.kernel(
        _body,
        out_type=(
            jax.ShapeDtypeStruct((TOTAL // CHUNK, CHUNK, EMBEDDING_DIM),
                                 jnp.float32),
            jax.ShapeDtypeStruct((NUM_EMBEDDINGS,), jnp.int32),
        ),
        mesh=mesh,
        compiler_params=pltpu.CompilerParams(use_tc_tiling_on_sc=False,
                                             needs_layout_passes=False),
        scratch_types=(
            pltpu.VMEM((2, NCHUNK, CHUNK), jnp.int32),
            pltpu.VMEM((NCHUNK, CHUNK), jnp.int32),
            pltpu.VMEM((2, CHUNK, 128), jnp.float32),
            pltpu.VMEM((2, CHUNK, EMBEDDING_DIM), jnp.float32),
            pltpu.VMEM((128,), jnp.int32),
            pltpu.VMEM((ZBUF,), jnp.int32),
            pltpu.VMEM_SHARED((CPAD,), jnp.int32),
            pltpu.SemaphoreType.DMA,
            pltpu.SemaphoreType.DMA,
            pltpu.SemaphoreType.DMA,
            pltpu.SemaphoreType.DMA,
        ),
    )
    return fn(idx, weight)


TW = 8192
TGRID = NBLK // TW


def _relayout_body(i0, i1, i2, i3, o_ref):
    o_ref[...] = jnp.concatenate(
        [i0[...], i1[...], i2[...], i3[...]], axis=0).T


def _relayout_tc(wt):
    last = NUM_EMBEDDINGS // TW
    specs = [
        pl.BlockSpec((EMBEDDING_DIM, TW),
                     lambda g, s=s: (0, jnp.minimum(g + s * TGRID, last)))
        for s in range(4)
    ]
    return pl.pallas_call(
        _relayout_body,
        grid=(TGRID,),
        in_specs=[specs[0], specs[1], specs[2], specs[3]],
        out_specs=pl.BlockSpec((TW, 4 * EMBEDDING_DIM), lambda g: (g, 0)),
        out_shape=jax.ShapeDtypeStruct(
            (NBLK, 4 * EMBEDDING_DIM), jnp.float32),
    )(wt, wt, wt, wt)


def kernel(input, weight):
    idx = input.reshape(NW, NCHUNK, CHUNK)
    w128 = _relayout_tc(weight.T)
    emb_flat, counts = _run(idx, w128)
    emb = emb_flat.reshape(input.shape + (EMBEDDING_DIM,))
    return emb, counts


# speedup vs baseline: 1.9874x; 1.0467x over previous
import functools

import jax
import jax.numpy as jnp
from jax import lax
from jax.experimental import pallas as pl
from jax.experimental.pallas import tpu as pltpu
from jax.experimental.pallas import tpu_sc as plsc

NUM_EMBEDDINGS = 1000000
EMBEDDING_DIM = 32

NC = 2
NS = 16
NW = NC * NS

TOTAL = 4096 * 26
RPW = TOTAL // NW
CHUNK = 128
NCHUNK = RPW // CHUNK
NBLK = 262144

CPT = 62720
CPAD = CPT * NS
CPT_LAST = NUM_EMBEDDINGS - (NS - 1) * CPT
ZBUF = 3136


def _body(idx_hbm, table_hbm, out_emb, out_counts,
          idx_v, idxq_v, big_v, comp_v, ones_v, zbuf_v, counts_sh,
          gsem, osem, csem, zsem):
    c = lax.axis_index("c")
    s = lax.axis_index("s")
    wid = s * NC + c
    cbase = wid * NCHUNK

    pltpu.sync_copy(idx_hbm.at[wid], idx_v.at[0])

    def qfill(t, _):
        jj = t // 8
        g = t - jj * 8
        v = idx_v[0, jj, pl.ds(g * 16, 16)]
        idxq_v[jj, pl.ds(g * 16, 16)] = v - (v // NBLK) * NBLK
        return _
    lax.fori_loop(0, NCHUNK * 8, qfill, 0)

    def gfire(j, b):
        pltpu.async_copy(table_hbm.at[idxq_v.at[j]], big_v.at[b], gsem)

    gfire(0, 0)

    @pl.when(c == 0)
    def _counts():
        def fill(i, _):
            zbuf_v[pl.ds(i * 16, 16)] = jnp.zeros((16,), jnp.int32)
            return _
        lax.fori_loop(0, ZBUF // 16, fill, 0)

        def zfire(k, _):
            pltpu.sync_copy(zbuf_v,
                            counts_sh.at[pl.ds(s * CPT + k * ZBUF, ZBUF)])
            return _
        lax.fori_loop(0, CPT // ZBUF, zfire, 0)

        def ofill(i, _):
            ones_v[pl.ds(i * 16, 16)] = jnp.ones((16,), jnp.int32)
            return _
        lax.fori_loop(0, CHUNK // 16, ofill, 0)

        pltpu.sync_copy(idx_hbm.at[wid + 1], idx_v.at[1])

        plsc.subcore_barrier()

        def scat(t, _):
            p = t // NCHUNK
            j = t - p * NCHUNK
            pltpu.sync_copy(ones_v.at[pl.ds(0, CHUNK)],
                            counts_sh.at[idx_v.at[p, j]], add=True)
            return _
        lax.fori_loop(0, 2 * NCHUNK, scat, 0)

        plsc.subcore_barrier()

        @pl.when(s < NS - 1)
        def _full():
            pltpu.sync_copy(counts_sh.at[pl.ds(s * CPT, CPT)],
                            out_counts.at[pl.ds(s * CPT, CPT)])

        @pl.when(s == NS - 1)
        def _last():
            pltpu.sync_copy(counts_sh.at[pl.ds(s * CPT, CPT_LAST)],
                            out_counts.at[pl.ds(s * CPT, CPT_LAST)])

    lane = jax.lax.iota(jnp.int32, 16)
    for j in range(NCHUNK):
        b = j % 2
        if j + 1 < NCHUNK:
            gfire(j + 1, 1 - b)
        pltpu.make_async_copy(table_hbm.at[idxq_v.at[j]], big_v.at[b],
                              gsem).wait()
        if j >= 2:
            pltpu.make_async_copy(comp_v.at[b],
                                  out_emb.at[cbase + j - 2],
                                  osem if b == 0 else zsem).wait()

        bb = jnp.full((16,), b, jnp.int32)
        def extract(g, _, j=j, b=b, bb=bb):
            idxv = idx_v[0, j, pl.ds(g * 16, 16)]
            off = (idxv // NBLK) * 32
            rows = g * 16 + lane
            def inner(ci, _):
                v = plsc.load_gather(big_v, [bb, rows, off + ci])
                plsc.store_scatter(comp_v, [bb, rows,
                                            jnp.full((16,), 0, jnp.int32) + ci],
                                   v)
                return _
            lax.fori_loop(0, 32, inner, 0)
            return _
        lax.fori_loop(0, 8, extract, 0)
        pltpu.async_copy(comp_v.at[b], out_emb.at[cbase + j],
                         osem if b == 0 else zsem)
    for j in (NCHUNK - 2, NCHUNK - 1):
        b = j % 2
        pltpu.make_async_copy(comp_v.at[b], out_emb.at[cbase + j],
                              osem if b == 0 else zsem).wait()


@jax.jit
def _run(idx, weight):
    mesh = plsc.VectorSubcoreMesh(core_axis_name="c", subcore_axis_name="s")
    fn = pl.kernel(
        _body,
        out_type=(
            jax.ShapeDtypeStruct((TOTAL // CHUNK, CHUNK, EMBEDDING_DIM),
                                 jnp.float32),
            jax.ShapeDtypeStruct((NUM_EMBEDDINGS,), jnp.int32),
        ),
        mesh=mesh,
        compiler_params=pltpu.CompilerParams(use_tc_tiling_on_sc=False,
                                             needs_layout_passes=False),
        scratch_types=(
            pltpu.VMEM((2, NCHUNK, CHUNK), jnp.int32),
            pltpu.VMEM((NCHUNK, CHUNK), jnp.int32),
            pltpu.VMEM((2, CHUNK, 128), jnp.float32),
            pltpu.VMEM((2, CHUNK, EMBEDDING_DIM), jnp.float32),
            pltpu.VMEM((128,), jnp.int32),
            pltpu.VMEM((ZBUF,), jnp.int32),
            pltpu.VMEM_SHARED((CPAD,), jnp.int32),
            pltpu.SemaphoreType.DMA,
            pltpu.SemaphoreType.DMA,
            pltpu.SemaphoreType.DMA,
            pltpu.SemaphoreType.DMA,
        ),
    )
    return fn(idx, weight)


TW = 16384
TGRID = NBLK // TW


def _relayout_body(i0, i1, i2, i3, o_ref):
    o_ref[...] = jnp.concatenate(
        [i0[...], i1[...], i2[...], i3[...]], axis=0).T


def _relayout_tc(wt):
    last = NUM_EMBEDDINGS // TW
    specs = [
        pl.BlockSpec((EMBEDDING_DIM, TW),
                     lambda g, s=s: (0, jnp.minimum(g + s * TGRID, last)))
        for s in range(4)
    ]
    return pl.pallas_call(
        _relayout_body,
        grid=(TGRID,),
        in_specs=[specs[0], specs[1], specs[2], specs[3]],
        out_specs=pl.BlockSpec((TW, 4 * EMBEDDING_DIM), lambda g: (g, 0)),
        out_shape=jax.ShapeDtypeStruct(
            (NBLK, 4 * EMBEDDING_DIM), jnp.float32),
    )(wt, wt, wt, wt)


def kernel(input, weight):
    idx = input.reshape(NW, NCHUNK, CHUNK)
    w128 = _relayout_tc(weight.T)
    emb_flat, counts = _run(idx, w128)
    emb = emb_flat.reshape(input.shape + (EMBEDDING_DIM,))
    return emb, counts
